# Initial kernel scaffold; baseline (speedup 1.0000x reference)
import jax
import jax.numpy as jnp
from jax.experimental import pallas as pl


def kernel(index, value, W_row, b_row, W_col, b_col, W_glob, b_glob, W_self, b_self, W_out, b_out):
    raise NotImplementedError("write your pallas kernel here")


# speedup vs baseline: 6.1943x; 6.1943x over previous
import functools

import jax
import jax.numpy as jnp
from jax import lax
from jax.experimental import pallas as pl
from jax.experimental.pallas import tpu as pltpu
from jax.experimental.pallas import tpu_sc as plsc

B, E, N, D = 2, 160000, 10000, 128
NC, NS = 2, 16
CHUNK = 128
ROWS = E // CHUNK
ITERS = -(-ROWS // NS)
FB = 80
FBLKS = N // FB
FITERS = -(-FBLKS // NS)
CNTW = 16

_SC_MESH = plsc.VectorSubcoreMesh(core_axis_name="c", subcore_axis_name="s")


def _p1s_body(val, idxr, idxc, s_row, s_col, vbuf, ibuf, acc):
    c = lax.axis_index("c")
    s = lax.axis_index("s")
    zv = jnp.zeros((16,), jnp.float32)

    for idx_hbm, s_out in ((idxr, s_row), (idxc, s_col)):
        def zfill(i, carry):
            r = i // (D // 16)
            k = i % (D // 16)
            vbuf[r, pl.ds(k * 16, 16)] = zv
            return carry

        lax.fori_loop(0, FB * (D // 16), zfill, 0)

        def zero_body(j, carry):
            blk = s + NS * j

            @pl.when(blk < FBLKS)
            def _():
                pltpu.sync_copy(vbuf.at[pl.ds(0, FB), :],
                                acc.at[pl.ds(blk * FB, FB), :])

            return carry

        lax.fori_loop(0, FITERS, zero_body, 0)
        plsc.subcore_barrier()

        def chunk_body(j, carry):
            row = s + NS * j

            @pl.when(row < ROWS)
            def _():
                pltpu.sync_copy(val.at[c, pl.ds(row * CHUNK, CHUNK), :], vbuf)
                pltpu.sync_copy(idx_hbm.at[c, row], ibuf)
                pltpu.sync_copy(vbuf, acc.at[ibuf], add=True)

            return carry

        lax.fori_loop(0, ITERS, chunk_body, 0)
        plsc.subcore_barrier()

        def flush_body(j, carry):
            blk = s + NS * j

            @pl.when(blk < FBLKS)
            def _():
                pltpu.sync_copy(acc.at[pl.ds(blk * FB, FB), :],
                                vbuf.at[pl.ds(0, FB), :])
                pltpu.sync_copy(vbuf.at[pl.ds(0, FB), :],
                                s_out.at[c, pl.ds(blk * FB, FB), :])

            return carry

        lax.fori_loop(0, FITERS, flush_body, 0)
        plsc.subcore_barrier()


_p1s = functools.partial(
    pl.kernel,
    out_type=(
        jax.ShapeDtypeStruct((B, N, D), jnp.float32),
        jax.ShapeDtypeStruct((B, N, D), jnp.float32),
    ),
    mesh=_SC_MESH,
    scratch_types=[
        pltpu.VMEM((CHUNK, D), jnp.float32),
        pltpu.VMEM((CHUNK,), jnp.int32),
        pltpu.VMEM_SHARED((N, D), jnp.float32),
    ],
)(_p1s_body)


def _p1c_body(idxr, idxc, ones_hbm, c_row, c_col, vbuf, ibuf, ones_v, cnt):
    c = lax.axis_index("c")
    s = lax.axis_index("s")
    zv = jnp.zeros((16,), jnp.float32)
    pltpu.sync_copy(ones_hbm, ones_v)

    for idx_hbm, c_out in ((idxr, c_row), (idxc, c_col)):
        def zfill(i, carry):
            r = i // (D // 16)
            k = i % (D // 16)
            vbuf[r, pl.ds(k * 16, 16)] = zv
            return carry

        lax.fori_loop(0, FB * (D // 16), zfill, 0)

        def zero_body(j, carry):
            blk = s + NS * j

            @pl.when(blk < FBLKS)
            def _():
                pltpu.sync_copy(vbuf.at[pl.ds(0, FB), :],
                                cnt.at[pl.ds(blk * FB, FB), :])

            return carry

        lax.fori_loop(0, FITERS, zero_body, 0)
        plsc.subcore_barrier()

        def chunk_body(j, carry):
            row = s + NS * j

            @pl.when(row < ROWS)
            def _():
                pltpu.sync_copy(idx_hbm.at[c, row], ibuf)
                pltpu.sync_copy(ones_v, cnt.at[ibuf], add=True)

            return carry

        lax.fori_loop(0, ITERS, chunk_body, 0)
        plsc.subcore_barrier()

        def flush_body(j, carry):
            blk = s + NS * j

            @pl.when(blk < FBLKS)
            def _():
                pltpu.sync_copy(cnt.at[pl.ds(blk * FB, FB), :],
                                vbuf.at[pl.ds(0, FB), :])
                pltpu.sync_copy(vbuf.at[pl.ds(0, FB), :],
                                c_out.at[c, pl.ds(blk * FB, FB), :])

            return carry

        lax.fori_loop(0, FITERS, flush_body, 0)
        plsc.subcore_barrier()


_p1c = functools.partial(
    pl.kernel,
    out_type=(
        jax.ShapeDtypeStruct((B, N, D), jnp.float32),
        jax.ShapeDtypeStruct((B, N, D), jnp.float32),
    ),
    mesh=_SC_MESH,
    scratch_types=[
        pltpu.VMEM((CHUNK, D), jnp.float32),
        pltpu.VMEM((CHUNK,), jnp.int32),
        pltpu.VMEM((CHUNK, D), jnp.float32),
        pltpu.VMEM_SHARED((N, D), jnp.float32),
    ],
)(_p1c_body)


_NB = 2000


def _p2_body(sr_ref, cr_ref, sc_ref, cc_ref, ar_ref, ac_ref,
             tr_ref, tc_ref, ps_ref):
    n = pl.program_id(1)
    sr = sr_ref[0]
    cr = cr_ref[0][:, 0:1] + 1e-9
    sc = sc_ref[0]
    cc = cc_ref[0][:, 0:1] + 1e-9
    tr_ref[0] = jnp.dot(sr / cr, ar_ref[...],
                        preferred_element_type=jnp.float32)
    tc_ref[0] = jnp.dot(sc / cc, ac_ref[...],
                        preferred_element_type=jnp.float32)
    vsum = jnp.broadcast_to(jnp.sum(sr, axis=0, keepdims=True), (8, D))

    @pl.when(n == 0)
    def _():
        ps_ref[0] = vsum

    @pl.when(n != 0)
    def _():
        ps_ref[0] = ps_ref[0] + vsum


def _p2(s_row, c_row, s_col, c_col, a_row_t, a_col_t):
    return pl.pallas_call(
        _p2_body,
        grid=(B, N // _NB),
        in_specs=[
            pl.BlockSpec((1, _NB, D), lambda b, n: (b, n, 0)),
            pl.BlockSpec((1, _NB, D), lambda b, n: (b, n, 0)),
            pl.BlockSpec((1, _NB, D), lambda b, n: (b, n, 0)),
            pl.BlockSpec((1, _NB, D), lambda b, n: (b, n, 0)),
            pl.BlockSpec((D, D), lambda b, n: (0, 0)),
            pl.BlockSpec((D, D), lambda b, n: (0, 0)),
        ],
        out_specs=[
            pl.BlockSpec((1, _NB, D), lambda b, n: (b, n, 0)),
            pl.BlockSpec((1, _NB, D), lambda b, n: (b, n, 0)),
            pl.BlockSpec((1, 8, D), lambda b, n: (b, 0, 0)),
        ],
        out_shape=[
            jax.ShapeDtypeStruct((B, N, D), jnp.float32),
            jax.ShapeDtypeStruct((B, N, D), jnp.float32),
            jax.ShapeDtypeStruct((B, 8, D), jnp.float32),
        ],
    )(s_row, c_row, s_col, c_col, a_row_t, a_col_t)


def _p3a_body(tr, tc, idxr, idxc, out, ibuf, abuf, bbuf, sem):
    c = lax.axis_index("c")
    s = lax.axis_index("s")

    def chunk_body(j, carry):
        row = s + NS * j

        @pl.when(row < ROWS)
        def _():
            pltpu.sync_copy(idxr.at[c, row], ibuf)
            pltpu.async_copy(tr.at[ibuf], abuf, sem).wait()
            pltpu.sync_copy(idxc.at[c, row], ibuf)
            pltpu.async_copy(tc.at[ibuf], bbuf, sem).wait()

            def add_body(r, carry2):
                for k in range(D // 16):
                    sl = pl.ds(k * 16, 16)
                    abuf[r, sl] = abuf[r, sl] + bbuf[r, sl]
                return carry2

            lax.fori_loop(0, CHUNK, add_body, 0)
            pltpu.sync_copy(abuf, out.at[c, pl.ds(row * CHUNK, CHUNK), :])

        return carry

    lax.fori_loop(0, ITERS, chunk_body, 0)


_p3a = functools.partial(
    pl.kernel,
    out_type=jax.ShapeDtypeStruct((B, E, D), jnp.float32),
    mesh=_SC_MESH,
    scratch_types=[
        pltpu.VMEM((CHUNK,), jnp.int32),
        pltpu.VMEM((CHUNK, D), jnp.float32),
        pltpu.VMEM((CHUNK, D), jnp.float32),
        pltpu.SemaphoreType.DMA,
    ],
)(_p3a_body)


_BE = 2000


def _p3b_body(x_ref, ss_ref, a_ref, g_ref, o_ref):
    y = (jnp.dot(x_ref[0], a_ref[...], preferred_element_type=jnp.float32)
         + ss_ref[0] + g_ref[0])
    o_ref[0] = jnp.where(y >= 0, y, 0.01 * y)


def _p3b(value, ssum, a_self_t, g):
    return pl.pallas_call(
        _p3b_body,
        grid=(B, E // _BE),
        in_specs=[
            pl.BlockSpec((1, _BE, D), lambda b, e: (b, e, 0)),
            pl.BlockSpec((1, _BE, D), lambda b, e: (b, e, 0)),
            pl.BlockSpec((D, D), lambda b, e: (0, 0)),
            pl.BlockSpec((1, 1, D), lambda b, e: (b, 0, 0)),
        ],
        out_specs=pl.BlockSpec((1, _BE, D), lambda b, e: (b, e, 0)),
        out_shape=jax.ShapeDtypeStruct((B, E, D), jnp.float32),
    )(value, ssum, a_self_t, g[:, None, :])


def kernel(index, value, W_row, b_row, W_col, b_col, W_glob, b_glob,
           W_self, b_self, W_out, b_out):
    Wo1 = W_out[:, 0:D]
    Wo2 = W_out[:, D:2 * D]
    Wo3 = W_out[:, 2 * D:3 * D]
    Wo4 = W_out[:, 3 * D:4 * D]
    a_self_t = (Wo1 @ W_self).T
    a_row_t = (Wo2 @ W_row).T
    a_col_t = (Wo3 @ W_col).T
    bias = b_out + b_row @ Wo2.T + b_col @ Wo3.T + b_self @ Wo1.T

    idx_row = index[:, :, 0].reshape(B, ROWS, CHUNK)
    idx_col = index[:, :, 1].reshape(B, ROWS, CHUNK)

    s_row, s_col = _p1s(value, idx_row, idx_col)
    c_row, c_col = _p1c(idx_row, idx_col, jnp.ones((CHUNK, D), jnp.float32))

    tbl_row, tbl_col, psum = _p2(s_row, c_row, s_col, c_col,
                                 a_row_t, a_col_t)

    vmean = psum[:, 0, :] / E
    g = (vmean @ W_glob.T + b_glob) @ Wo4.T + bias

    off = (jnp.arange(B, dtype=jnp.int32) * N)[:, None, None]
    ssum = _p3a(tbl_row.reshape(B * N, D), tbl_col.reshape(B * N, D),
                idx_row + off, idx_col + off)

    out = _p3b(value, ssum, a_self_t, g)
    return (index, out)


# speedup vs baseline: 7.1708x; 1.1576x over previous
import functools

import jax
import jax.numpy as jnp
from jax import lax
from jax.experimental import pallas as pl
from jax.experimental.pallas import tpu as pltpu
from jax.experimental.pallas import tpu_sc as plsc

B, E, N, D = 2, 160000, 10000, 128
NC, NS = 2, 16
CHUNK = 128
ROWS = E // CHUNK
ITERS = -(-ROWS // NS)
ITERS2 = (ITERS + 1) // 2
FB = 80
FBLKS = N // FB
FITERS = -(-FBLKS // NS)

_SC_MESH = plsc.VectorSubcoreMesh(core_axis_name="c", subcore_axis_name="s")


def _p1s_body(val, idxr, idxc, s_row, s_col, vbuf, ibuf, acc):
    c = lax.axis_index("c")
    s = lax.axis_index("s")
    zv = jnp.zeros((16,), jnp.float32)

    for idx_hbm, s_out in ((idxr, s_row), (idxc, s_col)):
        def zfill(i, carry):
            r = i // (D // 16)
            k = i % (D // 16)
            vbuf[r, pl.ds(k * 16, 16)] = zv
            return carry

        lax.fori_loop(0, FB * (D // 16), zfill, 0)

        def zero_body(j, carry):
            blk = s + NS * j

            @pl.when(blk < FBLKS)
            def _():
                pltpu.sync_copy(vbuf.at[pl.ds(0, FB), :],
                                acc.at[pl.ds(blk * FB, FB), :])

            return carry

        lax.fori_loop(0, FITERS, zero_body, 0)
        plsc.subcore_barrier()

        def chunk_body(j, carry):
            row = s + NS * j

            @pl.when(row < ROWS)
            def _():
                pltpu.sync_copy(val.at[c, pl.ds(row * CHUNK, CHUNK), :], vbuf)
                pltpu.sync_copy(idx_hbm.at[c, row], ibuf)
                pltpu.sync_copy(vbuf, acc.at[ibuf], add=True)

            return carry

        lax.fori_loop(0, ITERS, chunk_body, 0)
        plsc.subcore_barrier()

        def flush_body(j, carry):
            blk = s + NS * j

            @pl.when(blk < FBLKS)
            def _():
                pltpu.sync_copy(acc.at[pl.ds(blk * FB, FB), :],
                                vbuf.at[pl.ds(0, FB), :])
                pltpu.sync_copy(vbuf.at[pl.ds(0, FB), :],
                                s_out.at[c, pl.ds(blk * FB, FB), :])

            return carry

        lax.fori_loop(0, FITERS, flush_body, 0)
        plsc.subcore_barrier()


_p1s = functools.partial(
    pl.kernel,
    out_type=(
        jax.ShapeDtypeStruct((B, N, D), jnp.float32),
        jax.ShapeDtypeStruct((B, N, D), jnp.float32),
    ),
    mesh=_SC_MESH,
    scratch_types=[
        pltpu.VMEM((CHUNK, D), jnp.float32),
        pltpu.VMEM((CHUNK,), jnp.int32),
        pltpu.VMEM_SHARED((N, D), jnp.float32),
    ],
)(_p1s_body)


def _p1c_body(idxr, idxc, ones_hbm, c_row, c_col, vbuf, ibuf, ones_v, cnt):
    c = lax.axis_index("c")
    s = lax.axis_index("s")
    zv = jnp.zeros((16,), jnp.float32)
    pltpu.sync_copy(ones_hbm, ones_v)

    for idx_hbm, c_out in ((idxr, c_row), (idxc, c_col)):
        def zfill(i, carry):
            r = i // (D // 16)
            k = i % (D // 16)
            vbuf[r, pl.ds(k * 16, 16)] = zv
            return carry

        lax.fori_loop(0, FB * (D // 16), zfill, 0)

        def zero_body(j, carry):
            blk = s + NS * j

            @pl.when(blk < FBLKS)
            def _():
                pltpu.sync_copy(vbuf.at[pl.ds(0, FB), :],
                                cnt.at[pl.ds(blk * FB, FB), :])

            return carry

        lax.fori_loop(0, FITERS, zero_body, 0)
        plsc.subcore_barrier()

        def chunk_body(j, carry):
            row = s + NS * j

            @pl.when(row < ROWS)
            def _():
                pltpu.sync_copy(idx_hbm.at[c, row], ibuf)
                pltpu.sync_copy(ones_v, cnt.at[ibuf], add=True)

            return carry

        lax.fori_loop(0, ITERS, chunk_body, 0)
        plsc.subcore_barrier()

        def flush_body(j, carry):
            blk = s + NS * j

            @pl.when(blk < FBLKS)
            def _():
                pltpu.sync_copy(cnt.at[pl.ds(blk * FB, FB), :],
                                vbuf.at[pl.ds(0, FB), :])
                pltpu.sync_copy(vbuf.at[pl.ds(0, FB), :],
                                c_out.at[c, pl.ds(blk * FB, FB), :])

            return carry

        lax.fori_loop(0, FITERS, flush_body, 0)
        plsc.subcore_barrier()


_p1c = functools.partial(
    pl.kernel,
    out_type=(
        jax.ShapeDtypeStruct((B, N, D), jnp.float32),
        jax.ShapeDtypeStruct((B, N, D), jnp.float32),
    ),
    mesh=_SC_MESH,
    scratch_types=[
        pltpu.VMEM((CHUNK, D), jnp.float32),
        pltpu.VMEM((CHUNK,), jnp.int32),
        pltpu.VMEM((CHUNK, D), jnp.float32),
        pltpu.VMEM_SHARED((N, D), jnp.float32),
    ],
)(_p1c_body)


_NB = 2000


def _p2_body(sr_ref, cr_ref, sc_ref, cc_ref, ar_ref, ac_ref,
             tr_ref, tc_ref, ps_ref):
    n = pl.program_id(1)
    sr = sr_ref[0]
    cr = cr_ref[0][:, 0:1] + 1e-9
    sc = sc_ref[0]
    cc = cc_ref[0][:, 0:1] + 1e-9
    tr_ref[0] = jnp.dot(sr / cr, ar_ref[...],
                        preferred_element_type=jnp.float32)
    tc_ref[0] = jnp.dot(sc / cc, ac_ref[...],
                        preferred_element_type=jnp.float32)
    vsum = jnp.broadcast_to(jnp.sum(sr, axis=0, keepdims=True), (8, D))

    @pl.when(n == 0)
    def _():
        ps_ref[0] = vsum

    @pl.when(n != 0)
    def _():
        ps_ref[0] = ps_ref[0] + vsum


def _p2(s_row, c_row, s_col, c_col, a_row_t, a_col_t):
    return pl.pallas_call(
        _p2_body,
        grid=(B, N // _NB),
        in_specs=[
            pl.BlockSpec((1, _NB, D), lambda b, n: (b, n, 0)),
            pl.BlockSpec((1, _NB, D), lambda b, n: (b, n, 0)),
            pl.BlockSpec((1, _NB, D), lambda b, n: (b, n, 0)),
            pl.BlockSpec((1, _NB, D), lambda b, n: (b, n, 0)),
            pl.BlockSpec((D, D), lambda b, n: (0, 0)),
            pl.BlockSpec((D, D), lambda b, n: (0, 0)),
        ],
        out_specs=[
            pl.BlockSpec((1, _NB, D), lambda b, n: (b, n, 0)),
            pl.BlockSpec((1, _NB, D), lambda b, n: (b, n, 0)),
            pl.BlockSpec((1, 8, D), lambda b, n: (b, 0, 0)),
        ],
        out_shape=[
            jax.ShapeDtypeStruct((B, N, D), jnp.float32),
            jax.ShapeDtypeStruct((B, N, D), jnp.float32),
            jax.ShapeDtypeStruct((B, 8, D), jnp.float32),
        ],
    )(s_row, c_row, s_col, c_col, a_row_t, a_col_t)


def _p3a_body(tr, tc, idxr, idxc, out,
              ibr0, ibr1, ibc0, ibc1, ab0, ab1, bb0, bb1,
              sa0, sa1, sb0, sb1, sw0, sw1):
    c = lax.axis_index("c")
    s = lax.axis_index("s")
    ibr = (ibr0, ibr1)
    ibc = (ibc0, ibc1)
    ab = (ab0, ab1)
    bb = (bb0, bb1)
    sa = (sa0, sa1)
    sb = (sb0, sb1)
    sw = (sw0, sw1)

    pltpu.sync_copy(idxr.at[c, s], ibr[0])
    pltpu.sync_copy(idxc.at[c, s], ibc[0])
    pltpu.async_copy(tr.at[ibr[0]], ab[0], sa[0])
    pltpu.async_copy(tc.at[ibc[0]], bb[0], sb[0])

    def outer(t2, carry):
        for u in (0, 1):
            v = 1 - u
            j2 = 2 * t2 + u
            row = s + NS * j2
            nrow = row + NS

            @pl.when(row < ROWS)
            def _():
                pltpu.make_async_copy(tr.at[ibr[u]], ab[u], sa[u]).wait()
                pltpu.make_async_copy(tc.at[ibc[u]], bb[u], sb[u]).wait()

            @pl.when(jnp.logical_and(row - NS >= 0, row - NS < ROWS))
            def _():
                pltpu.make_async_copy(
                    ab[v], out.at[c, pl.ds(0, CHUNK), :], sw[v]).wait()

            @pl.when(nrow < ROWS)
            def _():
                pltpu.sync_copy(idxr.at[c, nrow], ibr[v])
                pltpu.sync_copy(idxc.at[c, nrow], ibc[v])
                pltpu.async_copy(tr.at[ibr[v]], ab[v], sa[v])
                pltpu.async_copy(tc.at[ibc[v]], bb[v], sb[v])

            @pl.when(row < ROWS)
            def _():
                def add_body(r, carry2):
                    for k in range(D // 16):
                        sl = pl.ds(k * 16, 16)
                        ab[u][r, sl] = ab[u][r, sl] + bb[u][r, sl]
                    return carry2

                lax.fori_loop(0, CHUNK, add_body, 0)
                pltpu.async_copy(
                    ab[u], out.at[c, pl.ds(row * CHUNK, CHUNK), :], sw[u])

        return carry

    lax.fori_loop(0, ITERS2, outer, 0)


_p3a = functools.partial(
    pl.kernel,
    out_type=jax.ShapeDtypeStruct((B, E, D), jnp.float32),
    mesh=_SC_MESH,
    scratch_types=[
        pltpu.VMEM((CHUNK,), jnp.int32),
        pltpu.VMEM((CHUNK,), jnp.int32),
        pltpu.VMEM((CHUNK,), jnp.int32),
        pltpu.VMEM((CHUNK,), jnp.int32),
        pltpu.VMEM((CHUNK, D), jnp.float32),
        pltpu.VMEM((CHUNK, D), jnp.float32),
        pltpu.VMEM((CHUNK, D), jnp.float32),
        pltpu.VMEM((CHUNK, D), jnp.float32),
        pltpu.SemaphoreType.DMA,
        pltpu.SemaphoreType.DMA,
        pltpu.SemaphoreType.DMA,
        pltpu.SemaphoreType.DMA,
        pltpu.SemaphoreType.DMA,
        pltpu.SemaphoreType.DMA,
    ],
)(_p3a_body)


_BE = 2000


def _p3b_body(x_ref, ss_ref, a_ref, g_ref, o_ref):
    y = (jnp.dot(x_ref[0], a_ref[...], preferred_element_type=jnp.float32)
         + ss_ref[0] + g_ref[0])
    o_ref[0] = jnp.where(y >= 0, y, 0.01 * y)


def _p3b(value, ssum, a_self_t, g):
    return pl.pallas_call(
        _p3b_body,
        grid=(B, E // _BE),
        in_specs=[
            pl.BlockSpec((1, _BE, D), lambda b, e: (b, e, 0)),
            pl.BlockSpec((1, _BE, D), lambda b, e: (b, e, 0)),
            pl.BlockSpec((D, D), lambda b, e: (0, 0)),
            pl.BlockSpec((1, 1, D), lambda b, e: (b, 0, 0)),
        ],
        out_specs=pl.BlockSpec((1, _BE, D), lambda b, e: (b, e, 0)),
        out_shape=jax.ShapeDtypeStruct((B, E, D), jnp.float32),
    )(value, ssum, a_self_t, g[:, None, :])


def kernel(index, value, W_row, b_row, W_col, b_col, W_glob, b_glob,
           W_self, b_self, W_out, b_out):
    Wo1 = W_out[:, 0:D]
    Wo2 = W_out[:, D:2 * D]
    Wo3 = W_out[:, 2 * D:3 * D]
    Wo4 = W_out[:, 3 * D:4 * D]
    a_self_t = (Wo1 @ W_self).T
    a_row_t = (Wo2 @ W_row).T
    a_col_t = (Wo3 @ W_col).T
    bias = b_out + b_row @ Wo2.T + b_col @ Wo3.T + b_self @ Wo1.T

    idx_row = index[:, :, 0].reshape(B, ROWS, CHUNK)
    idx_col = index[:, :, 1].reshape(B, ROWS, CHUNK)

    s_row, s_col = _p1s(value, idx_row, idx_col)
    c_row, c_col = _p1c(idx_row, idx_col, jnp.ones((CHUNK, D), jnp.float32))

    tbl_row, tbl_col, psum = _p2(s_row, c_row, s_col, c_col,
                                 a_row_t, a_col_t)

    vmean = psum[:, 0, :] / E
    g = (vmean @ W_glob.T + b_glob) @ Wo4.T + bias

    off = (jnp.arange(B, dtype=jnp.int32) * N)[:, None, None]
    ssum = _p3a(tbl_row.reshape(B * N, D), tbl_col.reshape(B * N, D),
                idx_row + off, idx_col + off)

    out = _p3b(value, ssum, a_self_t, g)
    return (index, out)


# speedup vs baseline: 8.9742x; 1.2515x over previous
import functools

import jax
import jax.numpy as jnp
from jax import lax
from jax.experimental import pallas as pl
from jax.experimental.pallas import tpu as pltpu
from jax.experimental.pallas import tpu_sc as plsc

B, E, N, D = 2, 160000, 10000, 128
NC, NS = 2, 16
CHUNK = 128
ROWS = E // CHUNK
ITERS = -(-ROWS // NS)
ITERS2 = (ITERS + 1) // 2
FB = 80
FBLKS = N // FB
FITERS = -(-FBLKS // NS)

_SC_MESH = plsc.VectorSubcoreMesh(core_axis_name="c", subcore_axis_name="s")


def _p1s_body(val, idxr, idxc, s_row, s_col,
              vb0, vb1, ib0, ib1, sv0, sv1, si0, si1, ss0, ss1, acc):
    c = lax.axis_index("c")
    s = lax.axis_index("s")
    zv = jnp.zeros((16,), jnp.float32)
    vb = (vb0, vb1)
    ib = (ib0, ib1)
    sv = (sv0, sv1)
    si = (si0, si1)
    ss = (ss0, ss1)

    for idx_hbm, s_out in ((idxr, s_row), (idxc, s_col)):
        def zfill(i, carry):
            r = i // (D // 16)
            k = i % (D // 16)
            vb0[r, pl.ds(k * 16, 16)] = zv
            return carry

        lax.fori_loop(0, FB * (D // 16), zfill, 0)

        def zero_body(j, carry):
            blk = s + NS * j

            @pl.when(blk < FBLKS)
            def _():
                pltpu.sync_copy(vb0.at[pl.ds(0, FB), :],
                                acc.at[pl.ds(blk * FB, FB), :])

            return carry

        lax.fori_loop(0, FITERS, zero_body, 0)
        plsc.subcore_barrier()

        pltpu.async_copy(val.at[c, pl.ds(s * CHUNK, CHUNK), :], vb[0], sv[0])
        pltpu.async_copy(idx_hbm.at[c, s], ib[0], si[0])

        def chunk_outer(t2, carry):
            for u in (0, 1):
                v = 1 - u
                j2 = 2 * t2 + u
                row = s + NS * j2
                nrow = row + NS

                @pl.when(row < ROWS)
                def _():
                    pltpu.make_async_copy(
                        val.at[c, pl.ds(row * CHUNK, CHUNK), :],
                        vb[u], sv[u]).wait()
                    pltpu.make_async_copy(
                        idx_hbm.at[c, row], ib[u], si[u]).wait()

                @pl.when(jnp.logical_and(row - NS >= 0, row - NS < ROWS))
                def _():
                    pltpu.make_async_copy(vb[v], acc.at[ib[v]],
                                          ss[v]).wait()

                @pl.when(nrow < ROWS)
                def _():
                    pltpu.async_copy(
                        val.at[c, pl.ds(nrow * CHUNK, CHUNK), :],
                        vb[v], sv[v])
                    pltpu.async_copy(idx_hbm.at[c, nrow], ib[v], si[v])

                @pl.when(row < ROWS)
                def _():
                    pltpu.async_copy(vb[u], acc.at[ib[u]], ss[u], add=True)

            return carry

        lax.fori_loop(0, ITERS2, chunk_outer, 0)
        plsc.subcore_barrier()

        def flush_body(j, carry):
            blk = s + NS * j

            @pl.when(blk < FBLKS)
            def _():
                pltpu.sync_copy(acc.at[pl.ds(blk * FB, FB), :],
                                vb0.at[pl.ds(0, FB), :])
                pltpu.sync_copy(vb0.at[pl.ds(0, FB), :],
                                s_out.at[c, pl.ds(blk * FB, FB), :])

            return carry

        lax.fori_loop(0, FITERS, flush_body, 0)
        plsc.subcore_barrier()


_p1s = functools.partial(
    pl.kernel,
    out_type=(
        jax.ShapeDtypeStruct((B, N, D), jnp.float32),
        jax.ShapeDtypeStruct((B, N, D), jnp.float32),
    ),
    mesh=_SC_MESH,
    scratch_types=[
        pltpu.VMEM((CHUNK, D), jnp.float32),
        pltpu.VMEM((CHUNK, D), jnp.float32),
        pltpu.VMEM((CHUNK,), jnp.int32),
        pltpu.VMEM((CHUNK,), jnp.int32),
        pltpu.SemaphoreType.DMA,
        pltpu.SemaphoreType.DMA,
        pltpu.SemaphoreType.DMA,
        pltpu.SemaphoreType.DMA,
        pltpu.SemaphoreType.DMA,
        pltpu.SemaphoreType.DMA,
        pltpu.VMEM_SHARED((N, D), jnp.float32),
    ],
)(_p1s_body)


def _p1c_body(idxr, idxc, ones_hbm, c_row, c_col,
              vbuf, ib0, ib1, si0, si1, ss0, ss1, ones_v, cnt):
    c = lax.axis_index("c")
    s = lax.axis_index("s")
    zv = jnp.zeros((16,), jnp.float32)
    ib = (ib0, ib1)
    si = (si0, si1)
    ss = (ss0, ss1)
    pltpu.sync_copy(ones_hbm, ones_v)

    for idx_hbm, c_out in ((idxr, c_row), (idxc, c_col)):
        def zfill(i, carry):
            r = i // (D // 16)
            k = i % (D // 16)
            vbuf[r, pl.ds(k * 16, 16)] = zv
            return carry

        lax.fori_loop(0, FB * (D // 16), zfill, 0)

        def zero_body(j, carry):
            blk = s + NS * j

            @pl.when(blk < FBLKS)
            def _():
                pltpu.sync_copy(vbuf.at[pl.ds(0, FB), :],
                                cnt.at[pl.ds(blk * FB, FB), :])

            return carry

        lax.fori_loop(0, FITERS, zero_body, 0)
        plsc.subcore_barrier()

        pltpu.async_copy(idx_hbm.at[c, s], ib[0], si[0])

        def chunk_outer(t2, carry):
            for u in (0, 1):
                v = 1 - u
                j2 = 2 * t2 + u
                row = s + NS * j2
                nrow = row + NS

                @pl.when(row < ROWS)
                def _():
                    pltpu.make_async_copy(
                        idx_hbm.at[c, row], ib[u], si[u]).wait()

                @pl.when(jnp.logical_and(row - NS >= 0, row - NS < ROWS))
                def _():
                    pltpu.make_async_copy(ones_v, cnt.at[ib[v]],
                                          ss[v]).wait()

                @pl.when(nrow < ROWS)
                def _():
                    pltpu.async_copy(idx_hbm.at[c, nrow], ib[v], si[v])

                @pl.when(row < ROWS)
                def _():
                    pltpu.async_copy(ones_v, cnt.at[ib[u]], ss[u], add=True)

            return carry

        lax.fori_loop(0, ITERS2, chunk_outer, 0)
        plsc.subcore_barrier()

        def flush_body(j, carry):
            blk = s + NS * j

            @pl.when(blk < FBLKS)
            def _():
                pltpu.sync_copy(cnt.at[pl.ds(blk * FB, FB), :],
                                vbuf.at[pl.ds(0, FB), :])
                pltpu.sync_copy(vbuf.at[pl.ds(0, FB), :],
                                c_out.at[c, pl.ds(blk * FB, FB), :])

            return carry

        lax.fori_loop(0, FITERS, flush_body, 0)
        plsc.subcore_barrier()


_p1c = functools.partial(
    pl.kernel,
    out_type=(
        jax.ShapeDtypeStruct((B, N, D), jnp.float32),
        jax.ShapeDtypeStruct((B, N, D), jnp.float32),
    ),
    mesh=_SC_MESH,
    scratch_types=[
        pltpu.VMEM((CHUNK, D), jnp.float32),
        pltpu.VMEM((CHUNK,), jnp.int32),
        pltpu.VMEM((CHUNK,), jnp.int32),
        pltpu.SemaphoreType.DMA,
        pltpu.SemaphoreType.DMA,
        pltpu.SemaphoreType.DMA,
        pltpu.SemaphoreType.DMA,
        pltpu.VMEM((CHUNK, D), jnp.float32),
        pltpu.VMEM_SHARED((N, D), jnp.float32),
    ],
)(_p1c_body)


_NB = 2000


def _p2_body(sr_ref, cr_ref, sc_ref, cc_ref, ar_ref, ac_ref,
             tr_ref, tc_ref, ps_ref):
    n = pl.program_id(1)
    sr = sr_ref[0]
    cr = cr_ref[0][:, 0:1] + 1e-9
    sc = sc_ref[0]
    cc = cc_ref[0][:, 0:1] + 1e-9
    tr_ref[0] = jnp.dot(sr / cr, ar_ref[...],
                        preferred_element_type=jnp.float32)
    tc_ref[0] = jnp.dot(sc / cc, ac_ref[...],
                        preferred_element_type=jnp.float32)
    vsum = jnp.broadcast_to(jnp.sum(sr, axis=0, keepdims=True), (8, D))

    @pl.when(n == 0)
    def _():
        ps_ref[0] = vsum

    @pl.when(n != 0)
    def _():
        ps_ref[0] = ps_ref[0] + vsum


def _p2(s_row, c_row, s_col, c_col, a_row_t, a_col_t):
    return pl.pallas_call(
        _p2_body,
        grid=(B, N // _NB),
        in_specs=[
            pl.BlockSpec((1, _NB, D), lambda b, n: (b, n, 0)),
            pl.BlockSpec((1, _NB, D), lambda b, n: (b, n, 0)),
            pl.BlockSpec((1, _NB, D), lambda b, n: (b, n, 0)),
            pl.BlockSpec((1, _NB, D), lambda b, n: (b, n, 0)),
            pl.BlockSpec((D, D), lambda b, n: (0, 0)),
            pl.BlockSpec((D, D), lambda b, n: (0, 0)),
        ],
        out_specs=[
            pl.BlockSpec((1, _NB, D), lambda b, n: (b, n, 0)),
            pl.BlockSpec((1, _NB, D), lambda b, n: (b, n, 0)),
            pl.BlockSpec((1, 8, D), lambda b, n: (b, 0, 0)),
        ],
        out_shape=[
            jax.ShapeDtypeStruct((B, N, D), jnp.float32),
            jax.ShapeDtypeStruct((B, N, D), jnp.float32),
            jax.ShapeDtypeStruct((B, 8, D), jnp.float32),
        ],
    )(s_row, c_row, s_col, c_col, a_row_t, a_col_t)


def _p3a_body(tr, tc, idxr, idxc, out,
              ibr0, ibr1, ibc0, ibc1, ab0, ab1, bb0, bb1,
              sa0, sa1, sb0, sb1, sw0, sw1):
    c = lax.axis_index("c")
    s = lax.axis_index("s")
    ibr = (ibr0, ibr1)
    ibc = (ibc0, ibc1)
    ab = (ab0, ab1)
    bb = (bb0, bb1)
    sa = (sa0, sa1)
    sb = (sb0, sb1)
    sw = (sw0, sw1)

    pltpu.sync_copy(idxr.at[c, s], ibr[0])
    pltpu.sync_copy(idxc.at[c, s], ibc[0])
    pltpu.async_copy(tr.at[ibr[0]], ab[0], sa[0])
    pltpu.async_copy(tc.at[ibc[0]], bb[0], sb[0])

    def outer(t2, carry):
        for u in (0, 1):
            v = 1 - u
            j2 = 2 * t2 + u
            row = s + NS * j2
            nrow = row + NS

            @pl.when(row < ROWS)
            def _():
                pltpu.make_async_copy(tr.at[ibr[u]], ab[u], sa[u]).wait()
                pltpu.make_async_copy(tc.at[ibc[u]], bb[u], sb[u]).wait()

            @pl.when(jnp.logical_and(row - NS >= 0, row - NS < ROWS))
            def _():
                pltpu.make_async_copy(
                    ab[v], out.at[c, pl.ds(0, CHUNK), :], sw[v]).wait()

            @pl.when(nrow < ROWS)
            def _():
                pltpu.sync_copy(idxr.at[c, nrow], ibr[v])
                pltpu.sync_copy(idxc.at[c, nrow], ibc[v])
                pltpu.async_copy(tr.at[ibr[v]], ab[v], sa[v])
                pltpu.async_copy(tc.at[ibc[v]], bb[v], sb[v])

            @pl.when(row < ROWS)
            def _():
                def add_body(r, carry2):
                    for k in range(D // 16):
                        sl = pl.ds(k * 16, 16)
                        ab[u][r, sl] = ab[u][r, sl] + bb[u][r, sl]
                    return carry2

                lax.fori_loop(0, CHUNK, add_body, 0)
                pltpu.async_copy(
                    ab[u], out.at[c, pl.ds(row * CHUNK, CHUNK), :], sw[u])

        return carry

    lax.fori_loop(0, ITERS2, outer, 0)


_p3a = functools.partial(
    pl.kernel,
    out_type=jax.ShapeDtypeStruct((B, E, D), jnp.float32),
    mesh=_SC_MESH,
    scratch_types=[
        pltpu.VMEM((CHUNK,), jnp.int32),
        pltpu.VMEM((CHUNK,), jnp.int32),
        pltpu.VMEM((CHUNK,), jnp.int32),
        pltpu.VMEM((CHUNK,), jnp.int32),
        pltpu.VMEM((CHUNK, D), jnp.float32),
        pltpu.VMEM((CHUNK, D), jnp.float32),
        pltpu.VMEM((CHUNK, D), jnp.float32),
        pltpu.VMEM((CHUNK, D), jnp.float32),
        pltpu.SemaphoreType.DMA,
        pltpu.SemaphoreType.DMA,
        pltpu.SemaphoreType.DMA,
        pltpu.SemaphoreType.DMA,
        pltpu.SemaphoreType.DMA,
        pltpu.SemaphoreType.DMA,
    ],
)(_p3a_body)


_BE = 2000


def _p3b_body(x_ref, ss_ref, a_ref, g_ref, o_ref):
    y = (jnp.dot(x_ref[0], a_ref[...], preferred_element_type=jnp.float32)
         + ss_ref[0] + g_ref[0])
    o_ref[0] = jnp.where(y >= 0, y, 0.01 * y)


def _p3b(value, ssum, a_self_t, g):
    return pl.pallas_call(
        _p3b_body,
        grid=(B, E // _BE),
        in_specs=[
            pl.BlockSpec((1, _BE, D), lambda b, e: (b, e, 0)),
            pl.BlockSpec((1, _BE, D), lambda b, e: (b, e, 0)),
            pl.BlockSpec((D, D), lambda b, e: (0, 0)),
            pl.BlockSpec((1, 1, D), lambda b, e: (b, 0, 0)),
        ],
        out_specs=pl.BlockSpec((1, _BE, D), lambda b, e: (b, e, 0)),
        out_shape=jax.ShapeDtypeStruct((B, E, D), jnp.float32),
    )(value, ssum, a_self_t, g[:, None, :])


def kernel(index, value, W_row, b_row, W_col, b_col, W_glob, b_glob,
           W_self, b_self, W_out, b_out):
    Wo1 = W_out[:, 0:D]
    Wo2 = W_out[:, D:2 * D]
    Wo3 = W_out[:, 2 * D:3 * D]
    Wo4 = W_out[:, 3 * D:4 * D]
    a_self_t = (Wo1 @ W_self).T
    a_row_t = (Wo2 @ W_row).T
    a_col_t = (Wo3 @ W_col).T
    bias = b_out + b_row @ Wo2.T + b_col @ Wo3.T + b_self @ Wo1.T

    idx_row = index[:, :, 0].reshape(B, ROWS, CHUNK)
    idx_col = index[:, :, 1].reshape(B, ROWS, CHUNK)

    s_row, s_col = _p1s(value, idx_row, idx_col)
    c_row, c_col = _p1c(idx_row, idx_col, jnp.ones((CHUNK, D), jnp.float32))

    tbl_row, tbl_col, psum = _p2(s_row, c_row, s_col, c_col,
                                 a_row_t, a_col_t)

    vmean = psum[:, 0, :] / E
    g = (vmean @ W_glob.T + b_glob) @ Wo4.T + bias

    off = (jnp.arange(B, dtype=jnp.int32) * N)[:, None, None]
    ssum = _p3a(tbl_row.reshape(B * N, D), tbl_col.reshape(B * N, D),
                idx_row + off, idx_col + off)

    out = _p3b(value, ssum, a_self_t, g)
    return (index, out)


# speedup vs baseline: 9.0129x; 1.0043x over previous
import functools

import jax
import jax.numpy as jnp
from jax import lax
from jax.experimental import pallas as pl
from jax.experimental.pallas import tpu as pltpu
from jax.experimental.pallas import tpu_sc as plsc

B, E, N, D = 2, 160000, 10000, 128
NC, NS = 2, 16
CHUNK = 128
ROWS = E // CHUNK
ITERS = -(-ROWS // NS)
ITERS2 = (ITERS + 1) // 2
FB = 80
FBLKS = N // FB
FITERS = -(-FBLKS // NS)

_SC_MESH = plsc.VectorSubcoreMesh(core_axis_name="c", subcore_axis_name="s")


def _p1_body(val, idxr, idxc, ones_hbm, s_row, s_col, c_row, c_col,
             vb0, vb1, ib0, ib1, sv0, sv1, si0, si1, ss0, ss1, acc):
    c = lax.axis_index("c")
    s = lax.axis_index("s")
    zv = jnp.zeros((16,), jnp.float32)
    vb = (vb0, vb1)
    ib = (ib0, ib1)
    sv = (sv0, sv1)
    si = (si0, si1)
    ss = (ss0, ss1)

    for idx_hbm, s_out in ((idxr, s_row), (idxc, s_col)):
        def zfill(i, carry):
            r = i // (D // 16)
            k = i % (D // 16)
            vb0[r, pl.ds(k * 16, 16)] = zv
            return carry

        lax.fori_loop(0, FB * (D // 16), zfill, 0)

        def zero_body(j, carry):
            blk = s + NS * j

            @pl.when(blk < FBLKS)
            def _():
                pltpu.sync_copy(vb0.at[pl.ds(0, FB), :],
                                acc.at[pl.ds(blk * FB, FB), :])

            return carry

        lax.fori_loop(0, FITERS, zero_body, 0)
        plsc.subcore_barrier()

        pltpu.async_copy(val.at[c, pl.ds(s * CHUNK, CHUNK), :], vb[0], sv[0])
        pltpu.async_copy(idx_hbm.at[c, s], ib[0], si[0])

        def chunk_outer(t2, carry):
            for u in (0, 1):
                v = 1 - u
                j2 = 2 * t2 + u
                row = s + NS * j2
                nrow = row + NS

                @pl.when(row < ROWS)
                def _():
                    pltpu.make_async_copy(
                        val.at[c, pl.ds(row * CHUNK, CHUNK), :],
                        vb[u], sv[u]).wait()
                    pltpu.make_async_copy(
                        idx_hbm.at[c, row], ib[u], si[u]).wait()

                @pl.when(jnp.logical_and(row - NS >= 0, row - NS < ROWS))
                def _():
                    pltpu.make_async_copy(vb[v], acc.at[ib[v]],
                                          ss[v]).wait()

                @pl.when(nrow < ROWS)
                def _():
                    pltpu.async_copy(
                        val.at[c, pl.ds(nrow * CHUNK, CHUNK), :],
                        vb[v], sv[v])
                    pltpu.async_copy(idx_hbm.at[c, nrow], ib[v], si[v])

                @pl.when(row < ROWS)
                def _():
                    pltpu.async_copy(vb[u], acc.at[ib[u]], ss[u], add=True)

            return carry

        lax.fori_loop(0, ITERS2, chunk_outer, 0)
        plsc.subcore_barrier()

        def flush_body(j, carry):
            blk = s + NS * j

            @pl.when(blk < FBLKS)
            def _():
                pltpu.sync_copy(acc.at[pl.ds(blk * FB, FB), :],
                                vb0.at[pl.ds(0, FB), :])
                pltpu.sync_copy(vb0.at[pl.ds(0, FB), :],
                                s_out.at[c, pl.ds(blk * FB, FB), :])

            return carry

        lax.fori_loop(0, FITERS, flush_body, 0)
        plsc.subcore_barrier()

    pltpu.sync_copy(ones_hbm, vb0)

    for idx_hbm, c_out in ((idxr, c_row), (idxc, c_col)):
        def zfillc(i, carry):
            r = i // (D // 16)
            k = i % (D // 16)
            vb1[r, pl.ds(k * 16, 16)] = zv
            return carry

        lax.fori_loop(0, FB * (D // 16), zfillc, 0)

        def zero_body_c(j, carry):
            blk = s + NS * j

            @pl.when(blk < FBLKS)
            def _():
                pltpu.sync_copy(vb1.at[pl.ds(0, FB), :],
                                acc.at[pl.ds(blk * FB, FB), :])

            return carry

        lax.fori_loop(0, FITERS, zero_body_c, 0)
        plsc.subcore_barrier()

        pltpu.async_copy(idx_hbm.at[c, s], ib[0], si[0])

        def chunk_outer_c(t2, carry):
            for u in (0, 1):
                v = 1 - u
                j2 = 2 * t2 + u
                row = s + NS * j2
                nrow = row + NS

                @pl.when(row < ROWS)
                def _():
                    pltpu.make_async_copy(
                        idx_hbm.at[c, row], ib[u], si[u]).wait()

                @pl.when(jnp.logical_and(row - NS >= 0, row - NS < ROWS))
                def _():
                    pltpu.make_async_copy(vb0, acc.at[ib[v]],
                                          ss[v]).wait()

                @pl.when(nrow < ROWS)
                def _():
                    pltpu.async_copy(idx_hbm.at[c, nrow], ib[v], si[v])

                @pl.when(row < ROWS)
                def _():
                    pltpu.async_copy(vb0, acc.at[ib[u]], ss[u], add=True)

            return carry

        lax.fori_loop(0, ITERS2, chunk_outer_c, 0)
        plsc.subcore_barrier()

        def flush_body_c(j, carry):
            blk = s + NS * j

            @pl.when(blk < FBLKS)
            def _():
                pltpu.sync_copy(acc.at[pl.ds(blk * FB, FB), :],
                                vb1.at[pl.ds(0, FB), :])
                pltpu.sync_copy(vb1.at[pl.ds(0, FB), :],
                                c_out.at[c, pl.ds(blk * FB, FB), :])

            return carry

        lax.fori_loop(0, FITERS, flush_body_c, 0)
        plsc.subcore_barrier()


_p1 = functools.partial(
    pl.kernel,
    out_type=(
        jax.ShapeDtypeStruct((B, N, D), jnp.float32),
        jax.ShapeDtypeStruct((B, N, D), jnp.float32),
        jax.ShapeDtypeStruct((B, N, D), jnp.float32),
        jax.ShapeDtypeStruct((B, N, D), jnp.float32),
    ),
    mesh=_SC_MESH,
    scratch_types=[
        pltpu.VMEM((CHUNK, D), jnp.float32),
        pltpu.VMEM((CHUNK, D), jnp.float32),
        pltpu.VMEM((CHUNK,), jnp.int32),
        pltpu.VMEM((CHUNK,), jnp.int32),
        pltpu.SemaphoreType.DMA,
        pltpu.SemaphoreType.DMA,
        pltpu.SemaphoreType.DMA,
        pltpu.SemaphoreType.DMA,
        pltpu.SemaphoreType.DMA,
        pltpu.SemaphoreType.DMA,
        pltpu.VMEM_SHARED((N, D), jnp.float32),
    ],
)(_p1_body)


_NB = 2000


def _p2_body(sr_ref, cr_ref, sc_ref, cc_ref, ar_ref, ac_ref,
             tr_ref, tc_ref, ps_ref):
    n = pl.program_id(1)
    sr = sr_ref[0]
    cr = cr_ref[0][:, 0:1] + 1e-9
    sc = sc_ref[0]
    cc = cc_ref[0][:, 0:1] + 1e-9
    tr_ref[0] = jnp.dot(sr / cr, ar_ref[...],
                        preferred_element_type=jnp.float32)
    tc_ref[0] = jnp.dot(sc / cc, ac_ref[...],
                        preferred_element_type=jnp.float32)
    vsum = jnp.broadcast_to(jnp.sum(sr, axis=0, keepdims=True), (8, D))

    @pl.when(n == 0)
    def _():
        ps_ref[0] = vsum

    @pl.when(n != 0)
    def _():
        ps_ref[0] = ps_ref[0] + vsum


def _p2(s_row, c_row, s_col, c_col, a_row_t, a_col_t):
    return pl.pallas_call(
        _p2_body,
        grid=(B, N // _NB),
        in_specs=[
            pl.BlockSpec((1, _NB, D), lambda b, n: (b, n, 0)),
            pl.BlockSpec((1, _NB, D), lambda b, n: (b, n, 0)),
            pl.BlockSpec((1, _NB, D), lambda b, n: (b, n, 0)),
            pl.BlockSpec((1, _NB, D), lambda b, n: (b, n, 0)),
            pl.BlockSpec((D, D), lambda b, n: (0, 0)),
            pl.BlockSpec((D, D), lambda b, n: (0, 0)),
        ],
        out_specs=[
            pl.BlockSpec((1, _NB, D), lambda b, n: (b, n, 0)),
            pl.BlockSpec((1, _NB, D), lambda b, n: (b, n, 0)),
            pl.BlockSpec((1, 8, D), lambda b, n: (b, 0, 0)),
        ],
        out_shape=[
            jax.ShapeDtypeStruct((B, N, D), jnp.float32),
            jax.ShapeDtypeStruct((B, N, D), jnp.float32),
            jax.ShapeDtypeStruct((B, 8, D), jnp.float32),
        ],
    )(s_row, c_row, s_col, c_col, a_row_t, a_col_t)


def _p3a_body(tr, tc, idxr, idxc, out,
              ibr0, ibr1, ibc0, ibc1, ab0, ab1, bb0, bb1,
              sa0, sa1, sb0, sb1, sw0, sw1):
    c = lax.axis_index("c")
    s = lax.axis_index("s")
    ibr = (ibr0, ibr1)
    ibc = (ibc0, ibc1)
    ab = (ab0, ab1)
    bb = (bb0, bb1)
    sa = (sa0, sa1)
    sb = (sb0, sb1)
    sw = (sw0, sw1)

    pltpu.sync_copy(idxr.at[c, s], ibr[0])
    pltpu.sync_copy(idxc.at[c, s], ibc[0])
    pltpu.async_copy(tr.at[ibr[0]], ab[0], sa[0])
    pltpu.async_copy(tc.at[ibc[0]], bb[0], sb[0])

    def outer(t2, carry):
        for u in (0, 1):
            v = 1 - u
            j2 = 2 * t2 + u
            row = s + NS * j2
            nrow = row + NS

            @pl.when(row < ROWS)
            def _():
                pltpu.make_async_copy(tr.at[ibr[u]], ab[u], sa[u]).wait()
                pltpu.make_async_copy(tc.at[ibc[u]], bb[u], sb[u]).wait()

            @pl.when(jnp.logical_and(row - NS >= 0, row - NS < ROWS))
            def _():
                pltpu.make_async_copy(
                    ab[v], out.at[c, pl.ds(0, CHUNK), :], sw[v]).wait()

            @pl.when(nrow < ROWS)
            def _():
                pltpu.sync_copy(idxr.at[c, nrow], ibr[v])
                pltpu.sync_copy(idxc.at[c, nrow], ibc[v])
                pltpu.async_copy(tr.at[ibr[v]], ab[v], sa[v])
                pltpu.async_copy(tc.at[ibc[v]], bb[v], sb[v])

            @pl.when(row < ROWS)
            def _():
                def add_body(r, carry2):
                    for k in range(D // 16):
                        sl = pl.ds(k * 16, 16)
                        ab[u][r, sl] = ab[u][r, sl] + bb[u][r, sl]
                    return carry2

                lax.fori_loop(0, CHUNK, add_body, 0)
                pltpu.async_copy(
                    ab[u], out.at[c, pl.ds(row * CHUNK, CHUNK), :], sw[u])

        return carry

    lax.fori_loop(0, ITERS2, outer, 0)


_p3a = functools.partial(
    pl.kernel,
    out_type=jax.ShapeDtypeStruct((B, E, D), jnp.float32),
    mesh=_SC_MESH,
    scratch_types=[
        pltpu.VMEM((CHUNK,), jnp.int32),
        pltpu.VMEM((CHUNK,), jnp.int32),
        pltpu.VMEM((CHUNK,), jnp.int32),
        pltpu.VMEM((CHUNK,), jnp.int32),
        pltpu.VMEM((CHUNK, D), jnp.float32),
        pltpu.VMEM((CHUNK, D), jnp.float32),
        pltpu.VMEM((CHUNK, D), jnp.float32),
        pltpu.VMEM((CHUNK, D), jnp.float32),
        pltpu.SemaphoreType.DMA,
        pltpu.SemaphoreType.DMA,
        pltpu.SemaphoreType.DMA,
        pltpu.SemaphoreType.DMA,
        pltpu.SemaphoreType.DMA,
        pltpu.SemaphoreType.DMA,
    ],
)(_p3a_body)


_BE = 2000


def _p3b_body(x_ref, ss_ref, a_ref, g_ref, o_ref):
    y = (jnp.dot(x_ref[0], a_ref[...], preferred_element_type=jnp.float32)
         + ss_ref[0] + g_ref[0])
    o_ref[0] = jnp.where(y >= 0, y, 0.01 * y)


def _p3b(value, ssum, a_self_t, g):
    return pl.pallas_call(
        _p3b_body,
        grid=(B, E // _BE),
        in_specs=[
            pl.BlockSpec((1, _BE, D), lambda b, e: (b, e, 0)),
            pl.BlockSpec((1, _BE, D), lambda b, e: (b, e, 0)),
            pl.BlockSpec((D, D), lambda b, e: (0, 0)),
            pl.BlockSpec((1, 1, D), lambda b, e: (b, 0, 0)),
        ],
        out_specs=pl.BlockSpec((1, _BE, D), lambda b, e: (b, e, 0)),
        out_shape=jax.ShapeDtypeStruct((B, E, D), jnp.float32),
    )(value, ssum, a_self_t, g[:, None, :])


def kernel(index, value, W_row, b_row, W_col, b_col, W_glob, b_glob,
           W_self, b_self, W_out, b_out):
    Wo1 = W_out[:, 0:D]
    Wo2 = W_out[:, D:2 * D]
    Wo3 = W_out[:, 2 * D:3 * D]
    Wo4 = W_out[:, 3 * D:4 * D]
    a_self_t = (Wo1 @ W_self).T
    a_row_t = (Wo2 @ W_row).T
    a_col_t = (Wo3 @ W_col).T
    bias = b_out + b_row @ Wo2.T + b_col @ Wo3.T + b_self @ Wo1.T

    idx_row = index[:, :, 0].reshape(B, ROWS, CHUNK)
    idx_col = index[:, :, 1].reshape(B, ROWS, CHUNK)

    s_row, s_col, c_row, c_col = _p1(value, idx_row, idx_col,
                                     jnp.ones((CHUNK, D), jnp.float32))

    tbl_row, tbl_col, psum = _p2(s_row, c_row, s_col, c_col,
                                 a_row_t, a_col_t)

    vmean = psum[:, 0, :] / E
    g = (vmean @ W_glob.T + b_glob) @ Wo4.T + bias

    off = (jnp.arange(B, dtype=jnp.int32) * N)[:, None, None]
    ssum = _p3a(tbl_row.reshape(B * N, D), tbl_col.reshape(B * N, D),
                idx_row + off, idx_col + off)

    out = _p3b(value, ssum, a_self_t, g)
    return (index, out)


# speedup vs baseline: 9.5005x; 1.0541x over previous
import functools

import jax
import jax.numpy as jnp
from jax import lax
from jax.experimental import pallas as pl
from jax.experimental.pallas import tpu as pltpu
from jax.experimental.pallas import tpu_sc as plsc

B, E, N, D = 2, 160000, 10000, 128
NC, NS = 2, 16
CHUNK = 128
ROWS = E // CHUNK
ITERS = -(-ROWS // NS)
ITERS2 = (ITERS + 1) // 2
FB = 80
FBLKS = N // FB
FITERS = -(-FBLKS // NS)

_SC_MESH = plsc.VectorSubcoreMesh(core_axis_name="c", subcore_axis_name="s")


def _p1_body(val, idxr, idxc, ones_hbm, s_row, s_col, c_row, c_col,
             vb0, vb1, ib0, ib1, sv0, sv1, si0, si1, ss0, ss1, acc):
    c = lax.axis_index("c")
    s = lax.axis_index("s")
    zv = jnp.zeros((16,), jnp.float32)
    vb = (vb0, vb1)
    ib = (ib0, ib1)
    sv = (sv0, sv1)
    si = (si0, si1)
    ss = (ss0, ss1)

    for idx_hbm, s_out in ((idxr, s_row), (idxc, s_col)):
        def zfill(i, carry):
            r = i // (D // 16)
            k = i % (D // 16)
            vb0[r, pl.ds(k * 16, 16)] = zv
            return carry

        lax.fori_loop(0, FB * (D // 16), zfill, 0)

        def zero_body(j, carry):
            blk = s + NS * j

            @pl.when(blk < FBLKS)
            def _():
                pltpu.sync_copy(vb0.at[pl.ds(0, FB), :],
                                acc.at[pl.ds(blk * FB, FB), :])

            return carry

        lax.fori_loop(0, FITERS, zero_body, 0)
        plsc.subcore_barrier()

        pltpu.async_copy(val.at[c, pl.ds(s * CHUNK, CHUNK), :], vb[0], sv[0])
        pltpu.async_copy(idx_hbm.at[c, s], ib[0], si[0])

        def chunk_outer(t2, carry):
            for u in (0, 1):
                v = 1 - u
                j2 = 2 * t2 + u
                row = s + NS * j2
                nrow = row + NS

                @pl.when(row < ROWS)
                def _():
                    pltpu.make_async_copy(
                        val.at[c, pl.ds(row * CHUNK, CHUNK), :],
                        vb[u], sv[u]).wait()
                    pltpu.make_async_copy(
                        idx_hbm.at[c, row], ib[u], si[u]).wait()

                @pl.when(jnp.logical_and(row - NS >= 0, row - NS < ROWS))
                def _():
                    pltpu.make_async_copy(vb[v], acc.at[ib[v]],
                                          ss[v]).wait()

                @pl.when(nrow < ROWS)
                def _():
                    pltpu.async_copy(
                        val.at[c, pl.ds(nrow * CHUNK, CHUNK), :],
                        vb[v], sv[v])
                    pltpu.async_copy(idx_hbm.at[c, nrow], ib[v], si[v])

                @pl.when(row < ROWS)
                def _():
                    pltpu.async_copy(vb[u], acc.at[ib[u]], ss[u], add=True)

            return carry

        lax.fori_loop(0, ITERS2, chunk_outer, 0)
        plsc.subcore_barrier()

        def flush_body(j, carry):
            blk = s + NS * j

            @pl.when(blk < FBLKS)
            def _():
                pltpu.sync_copy(acc.at[pl.ds(blk * FB, FB), :],
                                vb0.at[pl.ds(0, FB), :])
                pltpu.sync_copy(vb0.at[pl.ds(0, FB), :],
                                s_out.at[c, pl.ds(blk * FB, FB), :])

            return carry

        lax.fori_loop(0, FITERS, flush_body, 0)
        plsc.subcore_barrier()

    pltpu.sync_copy(ones_hbm, vb0)

    for idx_hbm, c_out in ((idxr, c_row), (idxc, c_col)):
        def zfillc(i, carry):
            r = i // (D // 16)
            k = i % (D // 16)
            vb1[r, pl.ds(k * 16, 16)] = zv
            return carry

        lax.fori_loop(0, FB * (D // 16), zfillc, 0)

        def zero_body_c(j, carry):
            blk = s + NS * j

            @pl.when(blk < FBLKS)
            def _():
                pltpu.sync_copy(vb1.at[pl.ds(0, FB), :],
                                acc.at[pl.ds(blk * FB, FB), :])

            return carry

        lax.fori_loop(0, FITERS, zero_body_c, 0)
        plsc.subcore_barrier()

        pltpu.async_copy(idx_hbm.at[c, s], ib[0], si[0])

        def chunk_outer_c(t2, carry):
            for u in (0, 1):
                v = 1 - u
                j2 = 2 * t2 + u
                row = s + NS * j2
                nrow = row + NS

                @pl.when(row < ROWS)
                def _():
                    pltpu.make_async_copy(
                        idx_hbm.at[c, row], ib[u], si[u]).wait()

                @pl.when(jnp.logical_and(row - NS >= 0, row - NS < ROWS))
                def _():
                    pltpu.make_async_copy(vb0, acc.at[ib[v]],
                                          ss[v]).wait()

                @pl.when(nrow < ROWS)
                def _():
                    pltpu.async_copy(idx_hbm.at[c, nrow], ib[v], si[v])

                @pl.when(row < ROWS)
                def _():
                    pltpu.async_copy(vb0, acc.at[ib[u]], ss[u], add=True)

            return carry

        lax.fori_loop(0, ITERS2, chunk_outer_c, 0)
        plsc.subcore_barrier()

        def flush_body_c(j, carry):
            blk = s + NS * j

            @pl.when(blk < FBLKS)
            def _():
                pltpu.sync_copy(acc.at[pl.ds(blk * FB, FB), :],
                                vb1.at[pl.ds(0, FB), :])
                pltpu.sync_copy(vb1.at[pl.ds(0, FB), :],
                                c_out.at[c, pl.ds(blk * FB, FB), :])

            return carry

        lax.fori_loop(0, FITERS, flush_body_c, 0)
        plsc.subcore_barrier()


_p1 = functools.partial(
    pl.kernel,
    out_type=(
        jax.ShapeDtypeStruct((B, N, D), jnp.float32),
        jax.ShapeDtypeStruct((B, N, D), jnp.float32),
        jax.ShapeDtypeStruct((B, N, D), jnp.float32),
        jax.ShapeDtypeStruct((B, N, D), jnp.float32),
    ),
    mesh=_SC_MESH,
    scratch_types=[
        pltpu.VMEM((CHUNK, D), jnp.float32),
        pltpu.VMEM((CHUNK, D), jnp.float32),
        pltpu.VMEM((CHUNK,), jnp.int32),
        pltpu.VMEM((CHUNK,), jnp.int32),
        pltpu.SemaphoreType.DMA,
        pltpu.SemaphoreType.DMA,
        pltpu.SemaphoreType.DMA,
        pltpu.SemaphoreType.DMA,
        pltpu.SemaphoreType.DMA,
        pltpu.SemaphoreType.DMA,
        pltpu.VMEM_SHARED((N, D), jnp.float32),
    ],
)(_p1_body)


_NB = 2000


def _p2_body(sr_ref, cr_ref, sc_ref, cc_ref, ar_ref, ac_ref,
             tr_ref, tc_ref, ps_ref):
    n = pl.program_id(1)
    sr = sr_ref[0]
    cr = cr_ref[0][:, 0:1] + 1e-9
    sc = sc_ref[0]
    cc = cc_ref[0][:, 0:1] + 1e-9
    tr_ref[0] = jnp.dot(sr / cr, ar_ref[...],
                        preferred_element_type=jnp.float32)
    tc_ref[0] = jnp.dot(sc / cc, ac_ref[...],
                        preferred_element_type=jnp.float32)
    vsum = jnp.broadcast_to(jnp.sum(sr, axis=0, keepdims=True), (8, D))

    @pl.when(n == 0)
    def _():
        ps_ref[0] = vsum

    @pl.when(n != 0)
    def _():
        ps_ref[0] = ps_ref[0] + vsum


def _p2(s_row, c_row, s_col, c_col, a_row_t, a_col_t):
    return pl.pallas_call(
        _p2_body,
        grid=(B, N // _NB),
        in_specs=[
            pl.BlockSpec((1, _NB, D), lambda b, n: (b, n, 0)),
            pl.BlockSpec((1, _NB, D), lambda b, n: (b, n, 0)),
            pl.BlockSpec((1, _NB, D), lambda b, n: (b, n, 0)),
            pl.BlockSpec((1, _NB, D), lambda b, n: (b, n, 0)),
            pl.BlockSpec((D, D), lambda b, n: (0, 0)),
            pl.BlockSpec((D, D), lambda b, n: (0, 0)),
        ],
        out_specs=[
            pl.BlockSpec((1, _NB, D), lambda b, n: (b, n, 0)),
            pl.BlockSpec((1, _NB, D), lambda b, n: (b, n, 0)),
            pl.BlockSpec((1, 8, D), lambda b, n: (b, 0, 0)),
        ],
        out_shape=[
            jax.ShapeDtypeStruct((B, N, D), jnp.float32),
            jax.ShapeDtypeStruct((B, N, D), jnp.float32),
            jax.ShapeDtypeStruct((B, 8, D), jnp.float32),
        ],
    )(s_row, c_row, s_col, c_col, a_row_t, a_col_t)


def _p3a_body(tr, tc, idxr, idxc, out,
              ibr0, ibr1, ibc0, ibc1, ab0, ab1, bb0, bb1,
              sa0, sa1, sb0, sb1, sw0, sw1):
    c = lax.axis_index("c")
    s = lax.axis_index("s")
    ibr = (ibr0, ibr1)
    ibc = (ibc0, ibc1)
    ab = (ab0, ab1)
    bb = (bb0, bb1)
    sa = (sa0, sa1)
    sb = (sb0, sb1)
    sw = (sw0, sw1)

    pltpu.sync_copy(idxr.at[c, s], ibr[0])
    pltpu.sync_copy(idxc.at[c, s], ibc[0])
    pltpu.async_copy(tr.at[ibr[0]], ab[0], sa[0])
    pltpu.async_copy(tc.at[ibc[0]], bb[0], sb[0])

    def outer(t2, carry):
        for u in (0, 1):
            v = 1 - u
            j2 = 2 * t2 + u
            row = s + NS * j2
            nrow = row + NS

            @pl.when(row < ROWS)
            def _():
                pltpu.make_async_copy(tr.at[ibr[u]], ab[u], sa[u]).wait()
                pltpu.make_async_copy(tc.at[ibc[u]], bb[u], sb[u]).wait()

            @pl.when(jnp.logical_and(row - NS >= 0, row - NS < ROWS))
            def _():
                pltpu.make_async_copy(
                    ab[v], out.at[c, pl.ds(0, CHUNK), :], sw[v]).wait()

            @pl.when(nrow < ROWS)
            def _():
                pltpu.sync_copy(idxr.at[c, nrow], ibr[v])
                pltpu.sync_copy(idxc.at[c, nrow], ibc[v])
                pltpu.async_copy(tr.at[ibr[v]], ab[v], sa[v])
                pltpu.async_copy(tc.at[ibc[v]], bb[v], sb[v])

            @pl.when(row < ROWS)
            def _():
                def add_body(r, carry2):
                    for k in range(D // 16):
                        sl = pl.ds(k * 16, 16)
                        ab[u][r, sl] = ab[u][r, sl] + bb[u][r, sl]
                    return carry2

                lax.fori_loop(0, CHUNK, add_body, 0)
                pltpu.async_copy(
                    ab[u], out.at[c, pl.ds(row * CHUNK, CHUNK), :], sw[u])

        return carry

    lax.fori_loop(0, ITERS2, outer, 0)


_p3a = functools.partial(
    pl.kernel,
    out_type=jax.ShapeDtypeStruct((B, E, D), jnp.float32),
    mesh=_SC_MESH,
    scratch_types=[
        pltpu.VMEM((CHUNK,), jnp.int32),
        pltpu.VMEM((CHUNK,), jnp.int32),
        pltpu.VMEM((CHUNK,), jnp.int32),
        pltpu.VMEM((CHUNK,), jnp.int32),
        pltpu.VMEM((CHUNK, D), jnp.float32),
        pltpu.VMEM((CHUNK, D), jnp.float32),
        pltpu.VMEM((CHUNK, D), jnp.float32),
        pltpu.VMEM((CHUNK, D), jnp.float32),
        pltpu.SemaphoreType.DMA,
        pltpu.SemaphoreType.DMA,
        pltpu.SemaphoreType.DMA,
        pltpu.SemaphoreType.DMA,
        pltpu.SemaphoreType.DMA,
        pltpu.SemaphoreType.DMA,
    ],
)(_p3a_body)


_BE = 5000


def _p3b_body(x_ref, ss_ref, a_ref, g_ref, o_ref):
    y = (jnp.dot(x_ref[0], a_ref[...], preferred_element_type=jnp.float32)
         + ss_ref[0] + g_ref[0])
    o_ref[0] = jnp.where(y >= 0, y, 0.01 * y)


def _p3b(value, ssum, a_self_t, g):
    return pl.pallas_call(
        _p3b_body,
        grid=(B, E // _BE),
        in_specs=[
            pl.BlockSpec((1, _BE, D), lambda b, e: (b, e, 0)),
            pl.BlockSpec((1, _BE, D), lambda b, e: (b, e, 0)),
            pl.BlockSpec((D, D), lambda b, e: (0, 0)),
            pl.BlockSpec((1, 1, D), lambda b, e: (b, 0, 0)),
        ],
        out_specs=pl.BlockSpec((1, _BE, D), lambda b, e: (b, e, 0)),
        out_shape=jax.ShapeDtypeStruct((B, E, D), jnp.float32),
    )(value, ssum, a_self_t, g[:, None, :])


def kernel(index, value, W_row, b_row, W_col, b_col, W_glob, b_glob,
           W_self, b_self, W_out, b_out):
    Wo1 = W_out[:, 0:D]
    Wo2 = W_out[:, D:2 * D]
    Wo3 = W_out[:, 2 * D:3 * D]
    Wo4 = W_out[:, 3 * D:4 * D]
    a_self_t = (Wo1 @ W_self).T
    a_row_t = (Wo2 @ W_row).T
    a_col_t = (Wo3 @ W_col).T
    bias = b_out + b_row @ Wo2.T + b_col @ Wo3.T + b_self @ Wo1.T

    idx_row = index[:, :, 0].reshape(B, ROWS, CHUNK)
    idx_col = index[:, :, 1].reshape(B, ROWS, CHUNK)

    s_row, s_col, c_row, c_col = _p1(value, idx_row, idx_col,
                                     jnp.ones((CHUNK, D), jnp.float32))

    tbl_row, tbl_col, psum = _p2(s_row, c_row, s_col, c_col,
                                 a_row_t, a_col_t)

    vmean = psum[:, 0, :] / E
    g = (vmean @ W_glob.T + b_glob) @ Wo4.T + bias

    off = (jnp.arange(B, dtype=jnp.int32) * N)[:, None, None]
    ssum = _p3a(tbl_row.reshape(B * N, D), tbl_col.reshape(B * N, D),
                idx_row + off, idx_col + off)

    out = _p3b(value, ssum, a_self_t, g)
    return (index, out)


# speedup vs baseline: 9.5958x; 1.0100x over previous
import functools

import jax
import jax.numpy as jnp
from jax import lax
from jax.experimental import pallas as pl
from jax.experimental.pallas import tpu as pltpu
from jax.experimental.pallas import tpu_sc as plsc

B, E, N, D = 2, 160000, 10000, 128
NC, NS = 2, 16
CHUNK = 128
ROWS = E // CHUNK
ITERS = -(-ROWS // NS)
ITERS2 = (ITERS + 1) // 2
FB = 80
FBLKS = N // FB
FITERS = -(-FBLKS // NS)

_SC_MESH = plsc.VectorSubcoreMesh(core_axis_name="c", subcore_axis_name="s")


def _p1_body(val, idxr, idxc, ones_hbm, s_row, s_col, c_row, c_col,
             vb0, vb1, ib0, ib1, sv0, sv1, si0, si1, ss0, ss1, acc):
    c = lax.axis_index("c")
    s = lax.axis_index("s")
    zv = jnp.zeros((16,), jnp.float32)
    vb = (vb0, vb1)
    ib = (ib0, ib1)
    sv = (sv0, sv1)
    si = (si0, si1)
    ss = (ss0, ss1)

    for idx_hbm, s_out in ((idxr, s_row), (idxc, s_col)):
        def zfill(i, carry):
            r = i // (D // 16)
            k = i % (D // 16)
            vb0[r, pl.ds(k * 16, 16)] = zv
            return carry

        lax.fori_loop(0, FB * (D // 16), zfill, 0)

        def zero_body(j, carry):
            blk = s + NS * j

            @pl.when(blk < FBLKS)
            def _():
                pltpu.sync_copy(vb0.at[pl.ds(0, FB), :],
                                acc.at[pl.ds(blk * FB, FB), :])

            return carry

        lax.fori_loop(0, FITERS, zero_body, 0)
        plsc.subcore_barrier()

        pltpu.async_copy(val.at[c, pl.ds(s * CHUNK, CHUNK), :], vb[0], sv[0])
        pltpu.async_copy(idx_hbm.at[c, s], ib[0], si[0])

        def chunk_outer(t2, carry):
            for u in (0, 1):
                v = 1 - u
                j2 = 2 * t2 + u
                row = s + NS * j2
                nrow = row + NS

                @pl.when(row < ROWS)
                def _():
                    pltpu.make_async_copy(
                        val.at[c, pl.ds(row * CHUNK, CHUNK), :],
                        vb[u], sv[u]).wait()
                    pltpu.make_async_copy(
                        idx_hbm.at[c, row], ib[u], si[u]).wait()

                @pl.when(jnp.logical_and(row - NS >= 0, row - NS < ROWS))
                def _():
                    pltpu.make_async_copy(vb[v], acc.at[ib[v]],
                                          ss[v]).wait()

                @pl.when(nrow < ROWS)
                def _():
                    pltpu.async_copy(
                        val.at[c, pl.ds(nrow * CHUNK, CHUNK), :],
                        vb[v], sv[v])
                    pltpu.async_copy(idx_hbm.at[c, nrow], ib[v], si[v])

                @pl.when(row < ROWS)
                def _():
                    pltpu.async_copy(vb[u], acc.at[ib[u]], ss[u], add=True)

            return carry

        lax.fori_loop(0, ITERS2, chunk_outer, 0)
        plsc.subcore_barrier()

        def flush_body(j, carry):
            blk = s + NS * j

            @pl.when(blk < FBLKS)
            def _():
                pltpu.sync_copy(acc.at[pl.ds(blk * FB, FB), :],
                                vb0.at[pl.ds(0, FB), :])
                pltpu.sync_copy(vb0.at[pl.ds(0, FB), :],
                                s_out.at[c, pl.ds(blk * FB, FB), :])

            return carry

        lax.fori_loop(0, FITERS, flush_body, 0)
        plsc.subcore_barrier()

    pltpu.sync_copy(ones_hbm, vb0)

    for idx_hbm, c_out in ((idxr, c_row), (idxc, c_col)):
        def zfillc(i, carry):
            r = i // (D // 16)
            k = i % (D // 16)
            vb1[r, pl.ds(k * 16, 16)] = zv
            return carry

        lax.fori_loop(0, FB * (D // 16), zfillc, 0)

        def zero_body_c(j, carry):
            blk = s + NS * j

            @pl.when(blk < FBLKS)
            def _():
                pltpu.sync_copy(vb1.at[pl.ds(0, FB), :],
                                acc.at[pl.ds(blk * FB, FB), :])

            return carry

        lax.fori_loop(0, FITERS, zero_body_c, 0)
        plsc.subcore_barrier()

        pltpu.async_copy(idx_hbm.at[c, s], ib[0], si[0])

        def chunk_outer_c(t2, carry):
            for u in (0, 1):
                v = 1 - u
                j2 = 2 * t2 + u
                row = s + NS * j2
                nrow = row + NS

                @pl.when(row < ROWS)
                def _():
                    pltpu.make_async_copy(
                        idx_hbm.at[c, row], ib[u], si[u]).wait()

                @pl.when(jnp.logical_and(row - NS >= 0, row - NS < ROWS))
                def _():
                    pltpu.make_async_copy(vb0, acc.at[ib[v]],
                                          ss[v]).wait()

                @pl.when(nrow < ROWS)
                def _():
                    pltpu.async_copy(idx_hbm.at[c, nrow], ib[v], si[v])

                @pl.when(row < ROWS)
                def _():
                    pltpu.async_copy(vb0, acc.at[ib[u]], ss[u], add=True)

            return carry

        lax.fori_loop(0, ITERS2, chunk_outer_c, 0)
        plsc.subcore_barrier()

        def flush_body_c(j, carry):
            blk = s + NS * j

            @pl.when(blk < FBLKS)
            def _():
                pltpu.sync_copy(acc.at[pl.ds(blk * FB, FB), :],
                                vb1.at[pl.ds(0, FB), :])
                pltpu.sync_copy(vb1.at[pl.ds(0, FB), :],
                                c_out.at[c, pl.ds(blk * FB, FB), :])

            return carry

        lax.fori_loop(0, FITERS, flush_body_c, 0)
        plsc.subcore_barrier()


_p1 = functools.partial(
    pl.kernel,
    out_type=(
        jax.ShapeDtypeStruct((B, N, D), jnp.float32),
        jax.ShapeDtypeStruct((B, N, D), jnp.float32),
        jax.ShapeDtypeStruct((B, N, D), jnp.float32),
        jax.ShapeDtypeStruct((B, N, D), jnp.float32),
    ),
    mesh=_SC_MESH,
    scratch_types=[
        pltpu.VMEM((CHUNK, D), jnp.float32),
        pltpu.VMEM((CHUNK, D), jnp.float32),
        pltpu.VMEM((CHUNK,), jnp.int32),
        pltpu.VMEM((CHUNK,), jnp.int32),
        pltpu.SemaphoreType.DMA,
        pltpu.SemaphoreType.DMA,
        pltpu.SemaphoreType.DMA,
        pltpu.SemaphoreType.DMA,
        pltpu.SemaphoreType.DMA,
        pltpu.SemaphoreType.DMA,
        pltpu.VMEM_SHARED((N, D), jnp.float32),
    ],
)(_p1_body)


_NB = 5000


def _p2_body(sr_ref, cr_ref, sc_ref, cc_ref, ar_ref, ac_ref,
             tr_ref, tc_ref, ps_ref):
    n = pl.program_id(1)
    sr = sr_ref[0]
    cr = cr_ref[0][:, 0:1] + 1e-9
    sc = sc_ref[0]
    cc = cc_ref[0][:, 0:1] + 1e-9
    tr_ref[0] = jnp.dot(sr / cr, ar_ref[...],
                        preferred_element_type=jnp.float32)
    tc_ref[0] = jnp.dot(sc / cc, ac_ref[...],
                        preferred_element_type=jnp.float32)
    vsum = jnp.broadcast_to(jnp.sum(sr, axis=0, keepdims=True), (8, D))

    @pl.when(n == 0)
    def _():
        ps_ref[0] = vsum

    @pl.when(n != 0)
    def _():
        ps_ref[0] = ps_ref[0] + vsum


def _p2(s_row, c_row, s_col, c_col, a_row_t, a_col_t):
    return pl.pallas_call(
        _p2_body,
        grid=(B, N // _NB),
        in_specs=[
            pl.BlockSpec((1, _NB, D), lambda b, n: (b, n, 0)),
            pl.BlockSpec((1, _NB, D), lambda b, n: (b, n, 0)),
            pl.BlockSpec((1, _NB, D), lambda b, n: (b, n, 0)),
            pl.BlockSpec((1, _NB, D), lambda b, n: (b, n, 0)),
            pl.BlockSpec((D, D), lambda b, n: (0, 0)),
            pl.BlockSpec((D, D), lambda b, n: (0, 0)),
        ],
        out_specs=[
            pl.BlockSpec((1, _NB, D), lambda b, n: (b, n, 0)),
            pl.BlockSpec((1, _NB, D), lambda b, n: (b, n, 0)),
            pl.BlockSpec((1, 8, D), lambda b, n: (b, 0, 0)),
        ],
        out_shape=[
            jax.ShapeDtypeStruct((B, N, D), jnp.float32),
            jax.ShapeDtypeStruct((B, N, D), jnp.float32),
            jax.ShapeDtypeStruct((B, 8, D), jnp.float32),
        ],
    )(s_row, c_row, s_col, c_col, a_row_t, a_col_t)


def _p3a_body(tr, tc, idxr, idxc, out,
              ibr0, ibr1, ibc0, ibc1, ab0, ab1, bb0, bb1,
              sa0, sa1, sb0, sb1, sw0, sw1):
    c = lax.axis_index("c")
    s = lax.axis_index("s")
    ibr = (ibr0, ibr1)
    ibc = (ibc0, ibc1)
    ab = (ab0, ab1)
    bb = (bb0, bb1)
    sa = (sa0, sa1)
    sb = (sb0, sb1)
    sw = (sw0, sw1)

    pltpu.sync_copy(idxr.at[c, s], ibr[0])
    pltpu.sync_copy(idxc.at[c, s], ibc[0])
    pltpu.async_copy(tr.at[ibr[0]], ab[0], sa[0])
    pltpu.async_copy(tc.at[ibc[0]], bb[0], sb[0])

    def outer(t2, carry):
        for u in (0, 1):
            v = 1 - u
            j2 = 2 * t2 + u
            row = s + NS * j2
            nrow = row + NS

            @pl.when(row < ROWS)
            def _():
                pltpu.make_async_copy(tr.at[ibr[u]], ab[u], sa[u]).wait()
                pltpu.make_async_copy(tc.at[ibc[u]], bb[u], sb[u]).wait()

            @pl.when(jnp.logical_and(row - NS >= 0, row - NS < ROWS))
            def _():
                pltpu.make_async_copy(
                    ab[v], out.at[c, pl.ds(0, CHUNK), :], sw[v]).wait()

            @pl.when(nrow < ROWS)
            def _():
                pltpu.sync_copy(idxr.at[c, nrow], ibr[v])
                pltpu.sync_copy(idxc.at[c, nrow], ibc[v])
                pltpu.async_copy(tr.at[ibr[v]], ab[v], sa[v])
                pltpu.async_copy(tc.at[ibc[v]], bb[v], sb[v])

            @pl.when(row < ROWS)
            def _():
                def add_body(r, carry2):
                    for k in range(D // 16):
                        sl = pl.ds(k * 16, 16)
                        ab[u][r, sl] = ab[u][r, sl] + bb[u][r, sl]
                    return carry2

                lax.fori_loop(0, CHUNK, add_body, 0)
                pltpu.async_copy(
                    ab[u], out.at[c, pl.ds(row * CHUNK, CHUNK), :], sw[u])

        return carry

    lax.fori_loop(0, ITERS2, outer, 0)


_p3a = functools.partial(
    pl.kernel,
    out_type=jax.ShapeDtypeStruct((B, E, D), jnp.float32),
    mesh=_SC_MESH,
    scratch_types=[
        pltpu.VMEM((CHUNK,), jnp.int32),
        pltpu.VMEM((CHUNK,), jnp.int32),
        pltpu.VMEM((CHUNK,), jnp.int32),
        pltpu.VMEM((CHUNK,), jnp.int32),
        pltpu.VMEM((CHUNK, D), jnp.float32),
        pltpu.VMEM((CHUNK, D), jnp.float32),
        pltpu.VMEM((CHUNK, D), jnp.float32),
        pltpu.VMEM((CHUNK, D), jnp.float32),
        pltpu.SemaphoreType.DMA,
        pltpu.SemaphoreType.DMA,
        pltpu.SemaphoreType.DMA,
        pltpu.SemaphoreType.DMA,
        pltpu.SemaphoreType.DMA,
        pltpu.SemaphoreType.DMA,
    ],
)(_p3a_body)


_BE = 10000


def _p3b_body(x_ref, ss_ref, a_ref, g_ref, o_ref):
    y = (jnp.dot(x_ref[0], a_ref[...], preferred_element_type=jnp.float32)
         + ss_ref[0] + g_ref[0])
    o_ref[0] = jnp.where(y >= 0, y, 0.01 * y)


def _p3b(value, ssum, a_self_t, g):
    return pl.pallas_call(
        _p3b_body,
        grid=(B, E // _BE),
        in_specs=[
            pl.BlockSpec((1, _BE, D), lambda b, e: (b, e, 0)),
            pl.BlockSpec((1, _BE, D), lambda b, e: (b, e, 0)),
            pl.BlockSpec((D, D), lambda b, e: (0, 0)),
            pl.BlockSpec((1, 1, D), lambda b, e: (b, 0, 0)),
        ],
        out_specs=pl.BlockSpec((1, _BE, D), lambda b, e: (b, e, 0)),
        out_shape=jax.ShapeDtypeStruct((B, E, D), jnp.float32),
    )(value, ssum, a_self_t, g[:, None, :])


def kernel(index, value, W_row, b_row, W_col, b_col, W_glob, b_glob,
           W_self, b_self, W_out, b_out):
    Wo1 = W_out[:, 0:D]
    Wo2 = W_out[:, D:2 * D]
    Wo3 = W_out[:, 2 * D:3 * D]
    Wo4 = W_out[:, 3 * D:4 * D]
    a_self_t = (Wo1 @ W_self).T
    a_row_t = (Wo2 @ W_row).T
    a_col_t = (Wo3 @ W_col).T
    bias = b_out + b_row @ Wo2.T + b_col @ Wo3.T + b_self @ Wo1.T

    idx_row = index[:, :, 0].reshape(B, ROWS, CHUNK)
    idx_col = index[:, :, 1].reshape(B, ROWS, CHUNK)

    s_row, s_col, c_row, c_col = _p1(value, idx_row, idx_col,
                                     jnp.ones((CHUNK, D), jnp.float32))

    tbl_row, tbl_col, psum = _p2(s_row, c_row, s_col, c_col,
                                 a_row_t, a_col_t)

    vmean = psum[:, 0, :] / E
    g = (vmean @ W_glob.T + b_glob) @ Wo4.T + bias

    off = (jnp.arange(B, dtype=jnp.int32) * N)[:, None, None]
    ssum = _p3a(tbl_row.reshape(B * N, D), tbl_col.reshape(B * N, D),
                idx_row + off, idx_col + off)

    out = _p3b(value, ssum, a_self_t, g)
    return (index, out)
